# baseline (speedup 1.0000x reference)
import functools

import jax
import jax.numpy as jnp
from jax import lax
from jax.experimental import pallas as pl
from jax.experimental.pallas import tpu as pltpu
from jax.experimental.pallas import tpu_sc as plsc

L = 16
CH = 128
SH = 7
NC = 2
NS = 16
NW = NC * NS
BB = 16


def _build(B, S, V, D):
    BT = (B // S) // NW
    T = BT * S
    TSUB = BB * S
    NSUB = BT // BB
    NVEC = TSUB // L
    NGR = (TSUB + CH - 1) // CH
    mesh = plsc.VectorSubcoreMesh(core_axis_name="c", subcore_axis_name="s")

    @functools.partial(
        pl.kernel,
        mesh=mesh,
        out_type=jax.ShapeDtypeStruct((S, D, B // S), jnp.float32),
        compiler_params=pltpu.CompilerParams(needs_layout_passes=False,
                                             use_tc_tiling_on_sc=False),
        scratch_types=[
            pltpu.VMEM((T,), jnp.int32),
            pltpu.VMEM((NGR, CH), jnp.int32),
            pltpu.VMEM((NGR * CH,), jnp.int32),
            pltpu.VMEM((NGR * CH, D), jnp.float32),
            pltpu.VMEM((S, D, BB), jnp.float32),
            pltpu.SemaphoreType.DMA,
        ],
    )
    def emb(ids_hbm, w_hbm, out_hbm, ids_v, grow, meta, rows, obuf, sem):
        wid = lax.axis_index("s") * NC + lax.axis_index("c")
        base = wid * T
        pltpu.sync_copy(ids_hbm.at[pl.ds(base, T)], ids_v)

        lanes = lax.iota(jnp.int32, L)
        zeros16 = jnp.zeros((L,), jnp.float32)
        zeros16i = jnp.zeros((L,), jnp.int32)

        def sub_chunk(c, carry):
            tbase = c * TSUB

            def clr(i, cc):
                for j in range(CH // L):
                    grow[i, pl.ds(j * L, L)] = zeros16i
                return cc
            lax.fori_loop(0, NGR, clr, 0)

            def compact(i, nin):
                ids16 = ids_v[pl.ds(tbase + i * L, L)]
                m = (ids16 >= 0) & (ids16 < V)
                mi = m.astype(jnp.int32)
                cs = jnp.cumsum(mi)
                s = jnp.sum(mi)
                p = jnp.maximum(nin + cs - 1, 0)
                plsc.store_scatter(grow, [p >> SH, p & (CH - 1)], ids16, mask=m)
                meta[pl.ds(i * L, L)] = jnp.where(m, nin + cs - 1, -1)
                return nin + s

            nin = lax.fori_loop(0, NVEC, compact, 0)

            ngr = (nin + CH - 1) >> SH

            @pl.when(ngr > 0)
            def _():
                def fire(g, cc):
                    pltpu.async_copy(w_hbm.at[grow.at[g]],
                                     rows.at[pl.ds(g * CH, CH)], sem)
                    return cc
                lax.fori_loop(0, ngr, fire, 0)

                def drain(g, cc):
                    pltpu.make_async_copy(w_hbm.at[grow.at[g]],
                                          rows.at[pl.ds(g * CH, CH)],
                                          sem).wait()
                    return cc
                lax.fori_loop(0, ngr, drain, 0)

            def zfill(i, cc):
                s_ = i // D
                d_ = i - s_ * D
                obuf[s_, d_, pl.ds(0, BB)] = zeros16
                return cc
            lax.fori_loop(0, S * D, zfill, 0)

            def dist(i, cc):
                mv = meta[pl.ds(i * L, L)]
                t0 = i * L
                for k in range(L):
                    r = mv[k]

                    @pl.when(r >= 0)
                    def _():
                        t = t0 + k
                        b_ = t // S
                        s_ = t - b_ * S
                        sv = jnp.full((L,), s_, jnp.int32)
                        bv = jnp.full((L,), b_, jnp.int32)
                        for q in range(D // L):
                            vals = rows[r, pl.ds(q * L, L)]
                            plsc.store_scatter(obuf, [sv, q * L + lanes, bv],
                                               vals)
                return cc
            lax.fori_loop(0, NVEC, dist, 0)

            bglob = wid * BT + c * BB
            pltpu.sync_copy(obuf, out_hbm.at[:, :, pl.ds(bglob, BB)])
            return carry

        lax.fori_loop(0, NSUB, sub_chunk, 0)

    return emb


@jax.jit
def kernel(input_ids, weight):
    B = input_ids.size
    S = input_ids.shape[-1]
    V, D = weight.shape
    ids_flat = input_ids.reshape(B)
    out_t = _build(B, S, V, D)(ids_flat, weight)
    return jnp.transpose(out_t, (2, 0, 1))


# speedup vs baseline: 2.0437x; 2.0437x over previous
import functools

import jax
import jax.numpy as jnp
from jax import lax
from jax.experimental import pallas as pl
from jax.experimental.pallas import tpu as pltpu
from jax.experimental.pallas import tpu_sc as plsc

L = 16
CH = 128
SH = 7
NC = 2
NS = 16
NW = NC * NS
NB = 4


def _build(B, V, D):
    assert B % (NW * L) == 0 and D % L == 0
    T = B // NW
    NVEC = T // L
    NR = T // CH + 2
    mesh = plsc.VectorSubcoreMesh(core_axis_name="c", subcore_axis_name="s")

    @functools.partial(
        pl.kernel,
        mesh=mesh,
        out_type=jax.ShapeDtypeStruct((B, D), jnp.float32),
        compiler_params=pltpu.CompilerParams(needs_layout_passes=False, use_tc_tiling_on_sc=False),
        scratch_types=[
            pltpu.VMEM((T,), jnp.int32),
            pltpu.VMEM((NR, CH), jnp.int32),
            pltpu.VMEM((NR, CH), jnp.int32),
            pltpu.VMEM((NR, CH), jnp.int32),
            pltpu.VMEM((NB * CH, D), jnp.float32),
            pltpu.VMEM((CH, D), jnp.float32),
        ] + [pltpu.SemaphoreType.DMA] * (2 * NB + 1),
    )
    def emb(ids_hbm, w_hbm, out_hbm, ids_v, inrow, indst, outdst, rowbuf,
            zbuf, *sems):
        semg = sems[:NB]
        sems_ = sems[NB:2 * NB]
        semz = sems[2 * NB]

        wid = lax.axis_index("s") * NC + lax.axis_index("c")
        base = wid * T
        pltpu.sync_copy(ids_hbm.at[pl.ds(base, T)], ids_v)

        def zrow(i, c):
            for j in range(D // L):
                zbuf[i, pl.ds(j * L, L)] = jnp.zeros((L,), jnp.float32)
            return c
        lax.fori_loop(0, CH, zrow, 0)

        lanes = lax.iota(jnp.int32, L)

        def compact(i, carry):
            nin, nout = carry
            ids16 = ids_v[pl.ds(i * L, L)]
            m = (ids16 >= 0) & (ids16 < V)
            mi = m.astype(jnp.int32)
            cs = jnp.cumsum(mi)
            s = jnp.sum(mi)
            pos = base + i * L + lanes
            pin = jnp.maximum(nin + cs - 1, 0)
            plsc.store_scatter(inrow, [pin >> SH, pin & (CH - 1)], ids16, mask=m)
            plsc.store_scatter(indst, [pin >> SH, pin & (CH - 1)], pos, mask=m)
            cso = jnp.cumsum(1 - mi)
            pout = jnp.maximum(nout + cso - 1, 0)
            plsc.store_scatter(outdst, [pout >> SH, pout & (CH - 1)], pos,
                               mask=jnp.logical_not(m))
            return nin + s, nout + (L - s)

        nin, nout = lax.fori_loop(0, NVEC, compact, (0, 0))

        def last_of(arr, n):
            q = jnp.full((L,), jnp.maximum(n - 1, 0), jnp.int32)
            return plsc.load_gather(arr, [q >> SH, q & (CH - 1)])

        def pad_tail(arr, n, val):
            for k in range(CH // L):
                p = n + k * L + lanes
                plsc.store_scatter(arr, [p >> SH, p & (CH - 1)], val)

        ncout = (nout + CH - 1) >> SH

        @pl.when(nout > 0)
        def _():
            pad_tail(outdst, nout, last_of(outdst, nout))

            def zfire(g, c):
                pltpu.async_copy(zbuf, out_hbm.at[outdst.at[g]], semz)
                return c
            lax.fori_loop(0, ncout, zfire, 0)

        @pl.when(nin > 0)
        def _():
            pad_tail(inrow, nin, last_of(inrow, nin))
            pad_tail(indst, nin, last_of(indst, nin))
            ncin = (nin + CH - 1) >> SH
            rounds = (ncin + NB - 1) // NB

            def round_body(r, c):
                for s in range(NB):
                    g = r * NB + s
                    buf = rowbuf.at[pl.ds(s * CH, CH)]

                    @pl.when(g < ncin)
                    def _():
                        @pl.when(r > 0)
                        def _():
                            pltpu.make_async_copy(
                                buf, out_hbm.at[indst.at[g]], sems_[s]).wait()
                        pltpu.async_copy(w_hbm.at[inrow.at[g]], buf, semg[s])
                for s in range(NB):
                    g = r * NB + s
                    buf = rowbuf.at[pl.ds(s * CH, CH)]

                    @pl.when(g < ncin)
                    def _():
                        pltpu.make_async_copy(
                            w_hbm.at[inrow.at[g]], buf, semg[s]).wait()
                        pltpu.async_copy(buf, out_hbm.at[indst.at[g]], sems_[s])
                return c
            lax.fori_loop(0, rounds, round_body, 0)

            for s in range(NB):
                @pl.when(s < ncin)
                def _():
                    pltpu.make_async_copy(
                        rowbuf.at[pl.ds(s * CH, CH)],
                        out_hbm.at[indst.at[0]], sems_[s]).wait()

        @pl.when(nout > 0)
        def _():
            def zdrain(g, c):
                pltpu.make_async_copy(zbuf, out_hbm.at[outdst.at[g]],
                                      semz).wait()
                return c
            lax.fori_loop(0, ncout, zdrain, 0)

    return emb


@jax.jit
def kernel(input_ids, weight):
    B = input_ids.size
    V, D = weight.shape
    ids_flat = input_ids.reshape(B)
    out = _build(B, V, D)(ids_flat, weight)
    return out.reshape(input_ids.shape + (D,))
